# Initial kernel scaffold; baseline (speedup 1.0000x reference)
import jax
import jax.numpy as jnp
from jax.experimental import pallas as pl


def kernel(x, edge_index, batch, idx_a, idx_b, W1, b1, W2, b2, fc1_W, fc1_b, fc_W, fc_b):
    raise NotImplementedError("write your pallas kernel here")


# speedup vs baseline: 2.9326x; 2.9326x over previous
import functools

import jax
import jax.numpy as jnp
from jax.experimental import pallas as pl
from jax.experimental.pallas import tpu as pltpu

_G = 1024
_BLK = 1024


def _pool_body(s_ref, b_ref, sums_ref, cnt_ref):
    i = pl.program_id(0)

    @pl.when(i == 0)
    def _init():
        sums_ref[...] = jnp.zeros_like(sums_ref)
        cnt_ref[...] = jnp.zeros_like(cnt_ref)

    gids = jax.lax.broadcasted_iota(jnp.int32, (1, _G), 1)
    onehot = (b_ref[...][:, 0][None, :, None] == gids[:, None, :]).astype(jnp.float32)
    oh = onehot[0]
    s = s_ref[...]
    sums_ref[...] += jax.lax.dot_general(
        s, oh, (((0,), (0,)), ((), ())),
        preferred_element_type=jnp.float32)
    cnt_ref[...] += jnp.sum(oh, axis=0, keepdims=True)


def _pool(s_pad, batch_pad, n_blocks):
    return pl.pallas_call(
        _pool_body,
        grid=(n_blocks,),
        in_specs=[
            pl.BlockSpec((_BLK, 1), lambda i: (i, 0)),
            pl.BlockSpec((_BLK, 1), lambda i: (i, 0)),
        ],
        out_specs=[
            pl.BlockSpec((1, _G), lambda i: (0, 0)),
            pl.BlockSpec((1, _G), lambda i: (0, 0)),
        ],
        out_shape=[
            jax.ShapeDtypeStruct((1, _G), jnp.float32),
            jax.ShapeDtypeStruct((1, _G), jnp.float32),
        ],
    )(s_pad, batch_pad)


def _gcn(h, src, dst, dinv, W, b):
    y = h * dinv[:, None]
    agg = jnp.zeros_like(y).at[dst].add(y[src]) + y
    return (agg * dinv[:, None]) @ W + b


def kernel(x, edge_index, batch, idx_a, idx_b, W1, b1, W2, b2,
           fc1_W, fc1_b, fc_W, fc_b):
    n = x.shape[0]
    src, dst = edge_index[0], edge_index[1]
    deg = jnp.zeros((n,), jnp.float32).at[dst].add(1.0) + 1.0
    dinv = jax.lax.rsqrt(deg)
    h = jnp.tanh(_gcn(x, src, dst, dinv, W1, b1))
    h = jnp.tanh(_gcn(h, src, dst, dinv, W2, b2))
    v = fc1_W @ fc_W
    c = fc1_b @ fc_W + fc_b
    s = (h @ v)[:, 0] + c[0]

    n_pad = ((n + _BLK - 1) // _BLK) * _BLK
    s_pad = jnp.pad(s, (0, n_pad - n)).reshape(n_pad, 1)
    batch_pad = jnp.pad(batch, (0, n_pad - n), constant_values=_G).reshape(n_pad, 1)
    sums, cnt = _pool(s_pad, batch_pad, n_pad // _BLK)
    util = (sums / jnp.maximum(cnt, 1.0))[0]
    return jnp.take(util, idx_b) - jnp.take(util, idx_a)


# speedup vs baseline: 33.5367x; 11.4358x over previous
import functools

import jax
import jax.numpy as jnp
from jax import lax
from jax.experimental import pallas as pl
from jax.experimental.pallas import tpu as pltpu
from jax.experimental.pallas import tpu_sc as plsc

_N = 50000
_NP = 50176
_G = 1024
_P = 8192
_BLK = 1024
_E = 1600000
_EP = 1605632
_CHUNK = 128
_NT = 32
_CPT = _EP // (_NT * _CHUNK)
_RPT = _NP // 16
_IC = 112


def _mesh():
    return plsc.VectorSubcoreMesh(core_axis_name="c", subcore_axis_name="s")


_SC_PARAMS = pltpu.CompilerParams(use_tc_tiling_on_sc=False)
_SC_PARAMS_NL = pltpu.CompilerParams(use_tc_tiling_on_sc=False,
                                     needs_layout_passes=False)


def _deg_build():
    @functools.partial(
        pl.kernel,
        out_type=jax.ShapeDtypeStruct((2 * _NP,), jnp.float32),
        mesh=_mesh(),
        compiler_params=_SC_PARAMS,
        scratch_types=[
            pltpu.VMEM_SHARED((_NP,), jnp.float32),
            pltpu.VMEM((_CHUNK,), jnp.int32),
            pltpu.VMEM((_CHUNK,), jnp.int32),
            pltpu.VMEM((_CHUNK,), jnp.float32),
            pltpu.VMEM((_RPT,), jnp.float32),
            pltpu.SemaphoreType.DMA,
            pltpu.SemaphoreType.DMA,
        ],
    )
    def k(dst_hbm, out_hbm, acc, d0, d1, onesb, initb, semd0, semd1):
        cid = lax.axis_index("c")
        sid = lax.axis_index("s")
        wid = cid * 16 + sid
        base = wid * _CPT
        r0 = sid * _RPT

        ones16 = jnp.full((16,), 1.0, jnp.float32)

        @pl.loop(0, _CHUNK, step=16)
        def _(i):
            onesb[pl.ds(i, 16)] = ones16

        @pl.loop(0, _RPT, step=16)
        def _(i):
            initb[pl.ds(i, 16)] = ones16

        pltpu.sync_copy(initb, acc.at[pl.ds(r0, _RPT)])
        plsc.subcore_barrier()

        didx = (d0, d1)
        sems = (semd0, semd1)

        def issue(g, b):
            pltpu.async_copy(dst_hbm.at[pl.ds((base + g) * _CHUNK, _CHUNK)],
                             didx[b], sems[b])

        def wait(g, b):
            pltpu.make_async_copy(
                dst_hbm.at[pl.ds((base + g) * _CHUNK, _CHUNK)],
                didx[b], sems[b]).wait()

        issue(0, 0)
        issue(1, 1)

        @pl.loop(0, _CPT, step=2)
        def _(g):
            for db in (0, 1):
                b = db
                gg = g + db
                wait(gg, b)
                pltpu.sync_copy(onesb, acc.at[didx[b]], add=True)

                @pl.when(gg + 2 < _CPT)
                def _():
                    issue(gg + 2, b)

        plsc.subcore_barrier()
        pltpu.sync_copy(acc.at[pl.ds(r0, _RPT)], initb)
        pltpu.sync_copy(initb, out_hbm.at[pl.ds(cid * _NP + r0, _RPT)])

    return k


def _agg_build(F):
    @functools.partial(
        pl.kernel,
        out_type=jax.ShapeDtypeStruct((2, _NP, F), jnp.float32),
        mesh=_mesh(),
        compiler_params=_SC_PARAMS,
        scratch_types=[
            pltpu.VMEM_SHARED((_NP, F), jnp.float32),
            pltpu.VMEM((_CHUNK,), jnp.int32),
            pltpu.VMEM((_CHUNK,), jnp.int32),
            pltpu.VMEM((_CHUNK,), jnp.int32),
            pltpu.VMEM((_CHUNK,), jnp.int32),
            pltpu.VMEM((_CHUNK, F), jnp.float32),
            pltpu.VMEM((_CHUNK, F), jnp.float32),
            pltpu.SemaphoreType.DMA,
            pltpu.SemaphoreType.DMA,
            pltpu.SemaphoreType.DMA,
            pltpu.SemaphoreType.DMA,
            pltpu.SemaphoreType.DMA,
            pltpu.SemaphoreType.DMA,
        ],
    )
    def k(src_hbm, dst_hbm, y_hbm, out_hbm, acc,
          s0, s1, d0, d1, r0buf, r1buf,
          ss0, ss1, sd0, sd1, sg0, sg1):
        cid = lax.axis_index("c")
        sid = lax.axis_index("s")
        wid = cid * 16 + sid
        base = wid * _CPT
        row0 = sid * _RPT

        sidx = (s0, s1)
        didx = (d0, d1)
        rows = (r0buf, r1buf)
        ssems = (ss0, ss1)
        dsems = (sd0, sd1)
        gsems = (sg0, sg1)

        @pl.loop(0, _RPT, step=_IC)
        def _(i):
            pltpu.sync_copy(y_hbm.at[pl.ds(row0 + i, _IC)],
                            rows[0].at[pl.ds(0, _IC)])
            pltpu.sync_copy(rows[0].at[pl.ds(0, _IC)],
                            acc.at[pl.ds(row0 + i, _IC)])

        plsc.subcore_barrier()

        def issue_idx(g, b):
            off = (base + g) * _CHUNK
            pltpu.async_copy(src_hbm.at[pl.ds(off, _CHUNK)], sidx[b], ssems[b])
            pltpu.async_copy(dst_hbm.at[pl.ds(off, _CHUNK)], didx[b], dsems[b])

        def wait_idx(g, b):
            off = (base + g) * _CHUNK
            pltpu.make_async_copy(src_hbm.at[pl.ds(off, _CHUNK)],
                                  sidx[b], ssems[b]).wait()
            pltpu.make_async_copy(dst_hbm.at[pl.ds(off, _CHUNK)],
                                  didx[b], dsems[b]).wait()

        def issue_gather(b):
            pltpu.async_copy(y_hbm.at[sidx[b]], rows[b], gsems[b])

        def wait_gather(b):
            pltpu.make_async_copy(y_hbm.at[sidx[b]], rows[b], gsems[b]).wait()

        issue_idx(0, 0)
        issue_idx(1, 1)
        wait_idx(0, 0)
        issue_gather(0)

        @pl.loop(0, _CPT, step=2)
        def _(g):
            for db in (0, 1):
                b = db
                nb = 1 - db
                gg = g + db
                wait_gather(b)

                @pl.when(gg + 1 < _CPT)
                def _():
                    wait_idx(gg + 1, nb)
                    issue_gather(nb)

                pltpu.sync_copy(rows[b], acc.at[didx[b]], add=True)

                @pl.when(gg + 2 < _CPT)
                def _():
                    issue_idx(gg + 2, b)

        plsc.subcore_barrier()

        @pl.loop(0, _RPT, step=_IC)
        def _(i):
            pltpu.sync_copy(acc.at[pl.ds(row0 + i, _IC)],
                            rows[0].at[pl.ds(0, _IC)])
            pltpu.sync_copy(rows[0].at[pl.ds(0, _IC)],
                            out_hbm.at[cid, pl.ds(row0 + i, _IC)])

    return k


def _pair_build():
    ppt = _P // _NT

    @functools.partial(
        pl.kernel,
        out_type=jax.ShapeDtypeStruct((_P,), jnp.float32),
        mesh=_mesh(),
        compiler_params=_SC_PARAMS_NL,
        scratch_types=[
            pltpu.VMEM((_G,), jnp.float32),
            pltpu.VMEM((_G,), jnp.float32),
            pltpu.VMEM((ppt,), jnp.int32),
            pltpu.VMEM((ppt,), jnp.int32),
            pltpu.VMEM((ppt,), jnp.float32),
            pltpu.SemaphoreType.DMA,
        ],
    )
    def k(sums_hbm, cnt_hbm, ia_hbm, ib_hbm, out_hbm,
          sbuf, ubuf, av, bv, ov, sem):
        cid = lax.axis_index("c")
        sid = lax.axis_index("s")
        wid = cid * 16 + sid
        p0 = wid * ppt

        pltpu.sync_copy(sums_hbm, sbuf)
        pltpu.sync_copy(cnt_hbm, ubuf)
        pltpu.sync_copy(ia_hbm.at[pl.ds(p0, ppt)], av)
        pltpu.sync_copy(ib_hbm.at[pl.ds(p0, ppt)], bv)

        one16 = jnp.full((16,), 1.0, jnp.float32)

        @pl.loop(0, _G, step=16)
        def _(i):
            sl = pl.ds(i, 16)
            ubuf[sl] = sbuf[sl] / jnp.maximum(ubuf[sl], one16)

        @pl.loop(0, ppt, step=16)
        def _(i):
            sl = pl.ds(i, 16)
            ua = plsc.load_gather(ubuf, [av[sl]])
            ub = plsc.load_gather(ubuf, [bv[sl]])
            ov[sl] = ub - ua

        pltpu.sync_copy(ov, out_hbm.at[pl.ds(p0, ppt)])

    return k


def _t1_body(p_ref, x_ref, dinv_ref, y1_ref):
    deg = p_ref[0] + p_ref[1] - 1.0
    dinv = lax.rsqrt(deg)
    dinv_ref[...] = dinv
    y1_ref[...] = x_ref[...] * dinv


def _t1(parts, x_pad):
    return pl.pallas_call(
        _t1_body,
        grid=(_NP // _BLK,),
        in_specs=[
            pl.BlockSpec((2, _BLK, 1), lambda i: (0, i, 0)),
            pl.BlockSpec((_BLK, 16), lambda i: (i, 0)),
        ],
        out_specs=[
            pl.BlockSpec((_BLK, 1), lambda i: (i, 0)),
            pl.BlockSpec((_BLK, 16), lambda i: (i, 0)),
        ],
        out_shape=[
            jax.ShapeDtypeStruct((_NP, 1), jnp.float32),
            jax.ShapeDtypeStruct((_NP, 16), jnp.float32),
        ],
    )(parts, x_pad)


def _t2_body(p_ref, y1_ref, dinv_ref, w_ref, b_ref, y2_ref):
    y1 = y1_ref[...]
    dinv = dinv_ref[...]
    agg = p_ref[0] + p_ref[1] - y1
    z = agg * dinv
    h = jnp.tanh(
        jax.lax.dot_general(z, w_ref[...], (((1,), (0,)), ((), ())),
                            precision=lax.Precision.HIGHEST,
                            preferred_element_type=jnp.float32)
        + b_ref[...])
    y2_ref[...] = h * dinv


def _t2(parts, y1, dinv, W1p, b1):
    return pl.pallas_call(
        _t2_body,
        grid=(_NP // _BLK,),
        in_specs=[
            pl.BlockSpec((2, _BLK, 16), lambda i: (0, i, 0)),
            pl.BlockSpec((_BLK, 16), lambda i: (i, 0)),
            pl.BlockSpec((_BLK, 1), lambda i: (i, 0)),
            pl.BlockSpec((16, 32), lambda i: (0, 0)),
            pl.BlockSpec((1, 32), lambda i: (0, 0)),
        ],
        out_specs=pl.BlockSpec((_BLK, 32), lambda i: (i, 0)),
        out_shape=jax.ShapeDtypeStruct((_NP, 32), jnp.float32),
    )(parts, y1, dinv, W1p, b1)


def _t3_body(p_ref, y2_ref, dinv_ref, w_ref, b_ref, v_ref, c_ref, batch_ref,
             sums_ref, cnt_ref):
    i = pl.program_id(0)

    @pl.when(i == 0)
    def _init():
        sums_ref[...] = jnp.zeros_like(sums_ref)
        cnt_ref[...] = jnp.zeros_like(cnt_ref)

    y2 = y2_ref[...]
    dinv = dinv_ref[...]
    agg = p_ref[0] + p_ref[1] - y2
    z = agg * dinv
    h = jnp.tanh(
        jax.lax.dot_general(z, w_ref[...], (((1,), (0,)), ((), ())),
                            precision=lax.Precision.HIGHEST,
                            preferred_element_type=jnp.float32)
        + b_ref[...])
    s = jax.lax.dot_general(h, v_ref[...], (((1,), (0,)), ((), ())),
                            precision=lax.Precision.HIGHEST,
                            preferred_element_type=jnp.float32) + c_ref[...]
    gids = jax.lax.broadcasted_iota(jnp.int32, (_BLK, _G), 1)
    oh = (batch_ref[...] == gids).astype(jnp.float32)
    sums_ref[...] += jax.lax.dot_general(
        s, oh, (((0,), (0,)), ((), ())),
        precision=lax.Precision.HIGHEST,
        preferred_element_type=jnp.float32)
    cnt_ref[...] += jnp.sum(oh, axis=0, keepdims=True)


def _t3(parts, y2, dinv, W2, b2, v, c, batch_pad):
    return pl.pallas_call(
        _t3_body,
        grid=(_NP // _BLK,),
        in_specs=[
            pl.BlockSpec((2, _BLK, 32), lambda i: (0, i, 0)),
            pl.BlockSpec((_BLK, 32), lambda i: (i, 0)),
            pl.BlockSpec((_BLK, 1), lambda i: (i, 0)),
            pl.BlockSpec((32, 32), lambda i: (0, 0)),
            pl.BlockSpec((1, 32), lambda i: (0, 0)),
            pl.BlockSpec((32, 1), lambda i: (0, 0)),
            pl.BlockSpec((1, 1), lambda i: (0, 0)),
            pl.BlockSpec((_BLK, 1), lambda i: (i, 0)),
        ],
        out_specs=[
            pl.BlockSpec((1, _G), lambda i: (0, 0)),
            pl.BlockSpec((1, _G), lambda i: (0, 0)),
        ],
        out_shape=[
            jax.ShapeDtypeStruct((1, _G), jnp.float32),
            jax.ShapeDtypeStruct((1, _G), jnp.float32),
        ],
    )(parts, y2, dinv, W2, b2, v, c, batch_pad)


def kernel(x, edge_index, batch, idx_a, idx_b, W1, b1, W2, b2,
           fc1_W, fc1_b, fc_W, fc_b):
    src = edge_index[0]
    dst = edge_index[1]
    pad_e = jnp.full((_EP - _E,), _N, jnp.int32)
    src_p = jnp.concatenate([src, pad_e])
    dst_p = jnp.concatenate([dst, pad_e])

    x_pad = jnp.pad(x, ((0, _NP - _N), (0, 16 - x.shape[1])))
    batch_pad = jnp.pad(batch, (0, _NP - _N),
                        constant_values=_G).reshape(_NP, 1)
    W1p = jnp.pad(W1, ((0, 16 - W1.shape[0]), (0, 0)))

    deg_parts = _deg_build()(dst_p)
    dinv, y1 = _t1(deg_parts.reshape(2, _NP, 1), x_pad)
    p1 = _agg_build(16)(src_p, dst_p, y1)
    y2 = _t2(p1, y1, dinv, W1p, b1.reshape(1, 32))
    p2 = _agg_build(32)(src_p, dst_p, y2)
    v = fc1_W @ fc_W
    c = (fc1_b @ fc_W + fc_b).reshape(1, 1)
    sums, cnt = _t3(p2, y2, dinv, W2, b2.reshape(1, 32), v, c, batch_pad)
    return _pair_build()(sums.reshape(_G), cnt.reshape(_G), idx_a, idx_b)


# speedup vs baseline: 49.2695x; 1.4691x over previous
import functools

import jax
import jax.numpy as jnp
from jax import lax
from jax.experimental import pallas as pl
from jax.experimental.pallas import tpu as pltpu
from jax.experimental.pallas import tpu_sc as plsc

_N = 50000
_NP = 50176
_G = 1024
_GB = 1280
_P = 8192
_BLK = 1024
_E = 1600000
_EP = 1605632
_CHUNK = 128
_NT = 32
_CPT = _EP // (_NT * _CHUNK)
_RPT = _NP // 16
_IC = 112


def _mesh():
    return plsc.VectorSubcoreMesh(core_axis_name="c", subcore_axis_name="s")


_SC_PARAMS = pltpu.CompilerParams(use_tc_tiling_on_sc=False)
_SC_PARAMS_NL = pltpu.CompilerParams(use_tc_tiling_on_sc=False,
                                     needs_layout_passes=False)


def _deg_build():
    @functools.partial(
        pl.kernel,
        out_type=jax.ShapeDtypeStruct((2 * _NP,), jnp.float32),
        mesh=_mesh(),
        compiler_params=_SC_PARAMS,
        scratch_types=[
            pltpu.VMEM_SHARED((_NP,), jnp.float32),
            pltpu.VMEM((4, _CHUNK), jnp.int32),
            pltpu.VMEM((_CHUNK,), jnp.float32),
            pltpu.VMEM((_RPT,), jnp.float32),
            pltpu.SemaphoreType.DMA((4,)),
            pltpu.SemaphoreType.DMA((4,)),
        ],
    )
    def k(dst_hbm, out_hbm, acc, didx, onesb, initb, isem, wsem):
        cid = lax.axis_index("c")
        sid = lax.axis_index("s")
        wid = cid * 16 + sid
        base = wid * _CPT
        r0 = sid * _RPT

        ones16 = jnp.full((16,), 1.0, jnp.float32)

        @pl.loop(0, _CHUNK, step=16)
        def _(i):
            onesb[pl.ds(i, 16)] = ones16

        @pl.loop(0, _RPT, step=16)
        def _(i):
            initb[pl.ds(i, 16)] = ones16

        pltpu.sync_copy(initb, acc.at[pl.ds(r0, _RPT)])
        plsc.subcore_barrier()

        def issue_idx(g, b):
            pltpu.async_copy(dst_hbm.at[pl.ds((base + g) * _CHUNK, _CHUNK)],
                             didx.at[b], isem.at[b])

        def wait_idx(g, b):
            pltpu.make_async_copy(
                dst_hbm.at[pl.ds((base + g) * _CHUNK, _CHUNK)],
                didx.at[b], isem.at[b]).wait()

        def issue_scatter(b):
            pltpu.async_copy(onesb, acc.at[didx.at[b]], wsem.at[b], add=True)

        def wait_scatter(b):
            pltpu.make_async_copy(onesb, acc.at[didx.at[b]], wsem.at[b]).wait()

        issue_idx(0, 0)
        issue_idx(1, 1)

        @pl.loop(0, _CPT, step=4)
        def _(g):
            for db in range(4):
                gg = g + db
                b = db % 4

                @pl.when(gg >= 2)
                def _():
                    wait_scatter((db + 2) % 4)

                @pl.when(gg + 2 < _CPT)
                def _():
                    issue_idx(gg + 2, (db + 2) % 4)

                wait_idx(gg, b)
                issue_scatter(b)

        wait_scatter(2)
        wait_scatter(3)

        plsc.subcore_barrier()
        pltpu.sync_copy(acc.at[pl.ds(r0, _RPT)], initb)
        pltpu.sync_copy(initb, out_hbm.at[pl.ds(cid * _NP + r0, _RPT)])

    return k


def _agg_build(F):
    @functools.partial(
        pl.kernel,
        out_type=jax.ShapeDtypeStruct((2, _NP, F), jnp.float32),
        mesh=_mesh(),
        compiler_params=_SC_PARAMS,
        scratch_types=[
            pltpu.VMEM_SHARED((_NP, F), jnp.float32),
            pltpu.VMEM((8, _CHUNK), jnp.int32),
            pltpu.VMEM((8, _CHUNK), jnp.int32),
            pltpu.VMEM((4, _CHUNK, F), jnp.float32),
            pltpu.SemaphoreType.DMA((8,)),
            pltpu.SemaphoreType.DMA((8,)),
            pltpu.SemaphoreType.DMA((4,)),
            pltpu.SemaphoreType.DMA((4,)),
        ],
    )
    def k(src_hbm, dst_hbm, y_hbm, out_hbm, acc,
          sidx, didx, rows, ssem, dsem, gsem, wsem):
        cid = lax.axis_index("c")
        sid = lax.axis_index("s")
        wid = cid * 16 + sid
        base = wid * _CPT
        row0 = sid * _RPT

        @pl.loop(0, _RPT, step=_IC)
        def _(i):
            pltpu.sync_copy(y_hbm.at[pl.ds(row0 + i, _IC)],
                            rows.at[0, pl.ds(0, _IC)])
            pltpu.sync_copy(rows.at[0, pl.ds(0, _IC)],
                            acc.at[pl.ds(row0 + i, _IC)])

        plsc.subcore_barrier()

        def issue_idx(g, b):
            off = (base + g) * _CHUNK
            pltpu.async_copy(src_hbm.at[pl.ds(off, _CHUNK)], sidx.at[b],
                             ssem.at[b])
            pltpu.async_copy(dst_hbm.at[pl.ds(off, _CHUNK)], didx.at[b],
                             dsem.at[b])

        def wait_idx(g, b):
            off = (base + g) * _CHUNK
            pltpu.make_async_copy(src_hbm.at[pl.ds(off, _CHUNK)],
                                  sidx.at[b], ssem.at[b]).wait()
            pltpu.make_async_copy(dst_hbm.at[pl.ds(off, _CHUNK)],
                                  didx.at[b], dsem.at[b]).wait()

        def issue_gather(bi, br):
            pltpu.async_copy(y_hbm.at[sidx.at[bi]], rows.at[br], gsem.at[br])

        def wait_gather(bi, br):
            pltpu.make_async_copy(y_hbm.at[sidx.at[bi]], rows.at[br],
                                  gsem.at[br]).wait()

        def issue_scatter(bi, br):
            pltpu.async_copy(rows.at[br], acc.at[didx.at[bi]], wsem.at[br],
                             add=True)

        def wait_scatter(bi, br):
            pltpu.make_async_copy(rows.at[br], acc.at[didx.at[bi]],
                                  wsem.at[br]).wait()

        for g0 in range(6):
            issue_idx(g0, g0)
        wait_idx(0, 0)
        issue_gather(0, 0)
        wait_idx(1, 1)
        issue_gather(1, 1)

        @pl.loop(0, _CPT, step=8)
        def _(g):
            for db in range(8):
                gg = g + db
                b8 = db % 8
                b4 = db % 4

                @pl.when(gg >= 2)
                def _():
                    wait_scatter((db + 6) % 8, (db + 2) % 4)

                @pl.when(gg + 6 < _CPT)
                def _():
                    issue_idx(gg + 6, (db + 6) % 8)

                @pl.when(gg + 2 < _CPT)
                def _():
                    wait_idx(gg + 2, (db + 2) % 8)
                    issue_gather((db + 2) % 8, (db + 2) % 4)

                wait_gather(b8, b4)
                issue_scatter(b8, b4)

        wait_scatter(6, 2)
        wait_scatter(7, 3)

        plsc.subcore_barrier()

        @pl.loop(0, _RPT, step=_IC)
        def _(i):
            pltpu.sync_copy(acc.at[pl.ds(row0 + i, _IC)],
                            rows.at[0, pl.ds(0, _IC)])
            pltpu.sync_copy(rows.at[0, pl.ds(0, _IC)],
                            out_hbm.at[cid, pl.ds(row0 + i, _IC)])

    return k


def _pool_pair_build():
    ppt = _P // _NT
    cols = _GB // 16

    @functools.partial(
        pl.kernel,
        out_type=jax.ShapeDtypeStruct((_P,), jnp.float32),
        mesh=_mesh(),
        compiler_params=_SC_PARAMS_NL,
        scratch_types=[
            pltpu.VMEM_SHARED((16, 2 * _GB), jnp.float32),
            pltpu.VMEM_SHARED((_GB,), jnp.float32),
            pltpu.VMEM((2 * _GB,), jnp.float32),
            pltpu.VMEM((_IC,), jnp.float32),
            pltpu.VMEM((_IC,), jnp.int32),
            pltpu.VMEM((16, cols), jnp.float32),
            pltpu.VMEM((_G,), jnp.float32),
            pltpu.VMEM((ppt,), jnp.int32),
            pltpu.VMEM((ppt,), jnp.int32),
            pltpu.VMEM((ppt,), jnp.float32),
            pltpu.SemaphoreType.DMA,
        ],
    )
    def k(s_hbm, batch_hbm, ia_hbm, ib_hbm, out_hbm,
          stage, ushared, hloc, sv, bv, comb, ubuf, av, bv2, ov, sem):
        cid = lax.axis_index("c")
        sid = lax.axis_index("s")
        wid = cid * 16 + sid
        r0 = sid * _RPT

        zero16 = jnp.zeros((16,), jnp.float32)
        one16 = jnp.full((16,), 1.0, jnp.float32)

        @pl.loop(0, 2 * _GB, step=16)
        def _(i):
            hloc[pl.ds(i, 16)] = zero16

        @pl.loop(0, _RPT, step=_IC)
        def _(i):
            pltpu.sync_copy(s_hbm.at[pl.ds(r0 + i, _IC)], sv)
            pltpu.sync_copy(batch_hbm.at[pl.ds(r0 + i, _IC)], bv)

            @pl.loop(0, _IC, step=16)
            def _(j):
                b16 = bv[pl.ds(j, 16)]
                plsc.addupdate_scatter(hloc, [b16], sv[pl.ds(j, 16)])
                plsc.addupdate_scatter(hloc, [b16 + _GB], one16)

        pltpu.sync_copy(hloc, stage.at[sid])
        plsc.subcore_barrier()

        c0 = sid * cols
        pltpu.sync_copy(stage.at[pl.ds(0, 16), pl.ds(c0, cols)], comb)

        @pl.loop(0, cols, step=16)
        def _(j):
            t = comb[0, pl.ds(j, 16)]
            for r in range(1, 16):
                t = t + comb[r, pl.ds(j, 16)]
            hloc[pl.ds(j, 16)] = t

        pltpu.sync_copy(stage.at[pl.ds(0, 16), pl.ds(_GB + c0, cols)], comb)

        @pl.loop(0, cols, step=16)
        def _(j):
            t = comb[0, pl.ds(j, 16)]
            for r in range(1, 16):
                t = t + comb[r, pl.ds(j, 16)]
            hloc[pl.ds(j, 16)] = hloc[pl.ds(j, 16)] / jnp.maximum(t, one16)

        pltpu.sync_copy(hloc.at[pl.ds(0, cols)], ushared.at[pl.ds(c0, cols)])
        plsc.subcore_barrier()

        pltpu.sync_copy(ushared.at[pl.ds(0, _G)], ubuf)

        p0 = wid * ppt
        pltpu.sync_copy(ia_hbm.at[pl.ds(p0, ppt)], av)
        pltpu.sync_copy(ib_hbm.at[pl.ds(p0, ppt)], bv2)

        @pl.loop(0, ppt, step=16)
        def _(i):
            sl = pl.ds(i, 16)
            ua = plsc.load_gather(ubuf, [av[sl]])
            ub = plsc.load_gather(ubuf, [bv2[sl]])
            ov[sl] = ub - ua

        pltpu.sync_copy(ov, out_hbm.at[pl.ds(p0, ppt)])

    return k


def _t1_body(p_ref, x_ref, dinv_ref, y1_ref):
    deg = p_ref[0] + p_ref[1] - 1.0
    dinv = lax.rsqrt(deg)
    dinv_ref[...] = dinv
    y1_ref[...] = x_ref[...] * dinv


def _t1(parts, x_pad):
    return pl.pallas_call(
        _t1_body,
        grid=(_NP // _BLK,),
        in_specs=[
            pl.BlockSpec((2, _BLK, 1), lambda i: (0, i, 0)),
            pl.BlockSpec((_BLK, 16), lambda i: (i, 0)),
        ],
        out_specs=[
            pl.BlockSpec((_BLK, 1), lambda i: (i, 0)),
            pl.BlockSpec((_BLK, 16), lambda i: (i, 0)),
        ],
        out_shape=[
            jax.ShapeDtypeStruct((_NP, 1), jnp.float32),
            jax.ShapeDtypeStruct((_NP, 16), jnp.float32),
        ],
    )(parts, x_pad)


def _t2_body(p_ref, y1_ref, dinv_ref, w_ref, b_ref, y2_ref):
    y1 = y1_ref[...]
    dinv = dinv_ref[...]
    agg = p_ref[0] + p_ref[1] - y1
    z = agg * dinv
    h = jnp.tanh(
        jax.lax.dot_general(z, w_ref[...], (((1,), (0,)), ((), ())),
                            precision=lax.Precision.HIGHEST,
                            preferred_element_type=jnp.float32)
        + b_ref[...])
    y2_ref[...] = h * dinv


def _t2(parts, y1, dinv, W1p, b1):
    return pl.pallas_call(
        _t2_body,
        grid=(_NP // _BLK,),
        in_specs=[
            pl.BlockSpec((2, _BLK, 16), lambda i: (0, i, 0)),
            pl.BlockSpec((_BLK, 16), lambda i: (i, 0)),
            pl.BlockSpec((_BLK, 1), lambda i: (i, 0)),
            pl.BlockSpec((16, 32), lambda i: (0, 0)),
            pl.BlockSpec((1, 32), lambda i: (0, 0)),
        ],
        out_specs=pl.BlockSpec((_BLK, 32), lambda i: (i, 0)),
        out_shape=jax.ShapeDtypeStruct((_NP, 32), jnp.float32),
    )(parts, y1, dinv, W1p, b1)


def _t3_body(p_ref, y2_ref, dinv_ref, w_ref, b_ref, f1w_ref, f1b_ref,
             fw_ref, fb_ref, s_ref):
    y2 = y2_ref[...]
    dinv = dinv_ref[...]
    agg = p_ref[0] + p_ref[1] - y2
    z = agg * dinv
    h = jnp.tanh(
        jax.lax.dot_general(z, w_ref[...], (((1,), (0,)), ((), ())),
                            precision=lax.Precision.HIGHEST,
                            preferred_element_type=jnp.float32)
        + b_ref[...])
    v = jax.lax.dot_general(f1w_ref[...], fw_ref[...], (((1,), (0,)), ((), ())),
                            precision=lax.Precision.HIGHEST,
                            preferred_element_type=jnp.float32)
    c = jax.lax.dot_general(f1b_ref[...], fw_ref[...], (((1,), (0,)), ((), ())),
                            precision=lax.Precision.HIGHEST,
                            preferred_element_type=jnp.float32) + fb_ref[...]
    s_ref[...] = jax.lax.dot_general(h, v, (((1,), (0,)), ((), ())),
                                     precision=lax.Precision.HIGHEST,
                                     preferred_element_type=jnp.float32) + c


def _t3(parts, y2, dinv, W2, b2, fc1_W, fc1_b, fc_W, fc_b):
    return pl.pallas_call(
        _t3_body,
        grid=(_NP // _BLK,),
        in_specs=[
            pl.BlockSpec((2, _BLK, 32), lambda i: (0, i, 0)),
            pl.BlockSpec((_BLK, 32), lambda i: (i, 0)),
            pl.BlockSpec((_BLK, 1), lambda i: (i, 0)),
            pl.BlockSpec((32, 32), lambda i: (0, 0)),
            pl.BlockSpec((1, 32), lambda i: (0, 0)),
            pl.BlockSpec((32, 32), lambda i: (0, 0)),
            pl.BlockSpec((1, 32), lambda i: (0, 0)),
            pl.BlockSpec((32, 1), lambda i: (0, 0)),
            pl.BlockSpec((1, 1), lambda i: (0, 0)),
        ],
        out_specs=pl.BlockSpec((_BLK, 1), lambda i: (i, 0)),
        out_shape=jax.ShapeDtypeStruct((_NP, 1), jnp.float32),
    )(parts, y2, dinv, W2, b2, fc1_W, fc1_b, fc_W, fc_b)


def kernel(x, edge_index, batch, idx_a, idx_b, W1, b1, W2, b2,
           fc1_W, fc1_b, fc_W, fc_b):
    src = edge_index[0]
    dst = edge_index[1]
    pad_e = jnp.full((_EP - _E,), _N, jnp.int32)
    src_p = jnp.concatenate([src, pad_e])
    dst_p = jnp.concatenate([dst, pad_e])

    x_pad = jnp.pad(x, ((0, _NP - _N), (0, 16 - x.shape[1])))
    batch_pad = jnp.pad(batch, (0, _NP - _N), constant_values=_G)
    W1p = jnp.pad(W1, ((0, 16 - W1.shape[0]), (0, 0)))

    deg_parts = _deg_build()(dst_p)
    dinv, y1 = _t1(deg_parts.reshape(2, _NP, 1), x_pad)
    p1 = _agg_build(16)(src_p, dst_p, y1)
    y2 = _t2(p1, y1, dinv, W1p, b1.reshape(1, 32))
    p2 = _agg_build(32)(src_p, dst_p, y2)
    s = _t3(p2, y2, dinv, W2, b2.reshape(1, 32),
            fc1_W, fc1_b.reshape(1, 32), fc_W, fc_b.reshape(1, 1))
    return _pool_pair_build()(s.reshape(_NP), batch_pad, idx_a, idx_b)


# speedup vs baseline: 76.7919x; 1.5586x over previous
import functools

import jax
import jax.numpy as jnp
from jax import lax
from jax.experimental import pallas as pl
from jax.experimental.pallas import tpu as pltpu
from jax.experimental.pallas import tpu_sc as plsc

_N = 50000
_NP = 50176
_G = 1024
_GB = 1280
_P = 8192
_BLK = 7168
_E = 1600000
_EP = 1605632
_CHUNK = 128
_NT = 32
_CPT = _EP // (_NT * _CHUNK)
_RPT = _NP // 16
_IC = 112


def _mesh():
    return plsc.VectorSubcoreMesh(core_axis_name="c", subcore_axis_name="s")


_SC_PARAMS = pltpu.CompilerParams(use_tc_tiling_on_sc=False)
_SC_PARAMS_NL = pltpu.CompilerParams(use_tc_tiling_on_sc=False,
                                     needs_layout_passes=False)


def _deg_build():
    @functools.partial(
        pl.kernel,
        out_type=jax.ShapeDtypeStruct((2 * _NP,), jnp.float32),
        mesh=_mesh(),
        compiler_params=_SC_PARAMS,
        scratch_types=[
            pltpu.VMEM_SHARED((_NP,), jnp.float32),
            pltpu.VMEM((4, _CHUNK), jnp.int32),
            pltpu.VMEM((_CHUNK,), jnp.float32),
            pltpu.VMEM((_RPT,), jnp.float32),
            pltpu.SemaphoreType.DMA((4,)),
            pltpu.SemaphoreType.DMA((4,)),
        ],
    )
    def k(ep_hbm, out_hbm, acc, didx, onesb, initb, isem, wsem):
        cid = lax.axis_index("c")
        sid = lax.axis_index("s")
        wid = cid * 16 + sid
        base = wid * _CPT
        r0 = sid * _RPT

        ones16 = jnp.full((16,), 1.0, jnp.float32)

        @pl.loop(0, _CHUNK, step=16)
        def _(i):
            onesb[pl.ds(i, 16)] = ones16

        @pl.loop(0, _RPT, step=16)
        def _(i):
            initb[pl.ds(i, 16)] = ones16

        pltpu.sync_copy(initb, acc.at[pl.ds(r0, _RPT)])
        plsc.subcore_barrier()

        def issue_idx(g, b):
            pltpu.async_copy(ep_hbm.at[1, pl.ds((base + g) * _CHUNK, _CHUNK)],
                             didx.at[b], isem.at[b])

        def wait_idx(g, b):
            pltpu.make_async_copy(
                ep_hbm.at[1, pl.ds((base + g) * _CHUNK, _CHUNK)],
                didx.at[b], isem.at[b]).wait()

        def issue_scatter(b):
            pltpu.async_copy(onesb, acc.at[didx.at[b]], wsem.at[b], add=True)

        def wait_scatter(b):
            pltpu.make_async_copy(onesb, acc.at[didx.at[b]], wsem.at[b]).wait()

        issue_idx(0, 0)
        issue_idx(1, 1)

        @pl.loop(0, _CPT, step=4)
        def _(g):
            for db in range(4):
                gg = g + db
                b = db % 4

                @pl.when(gg >= 2)
                def _():
                    wait_scatter((db + 2) % 4)

                @pl.when(gg + 2 < _CPT)
                def _():
                    issue_idx(gg + 2, (db + 2) % 4)

                wait_idx(gg, b)
                issue_scatter(b)

        wait_scatter(2)
        wait_scatter(3)

        plsc.subcore_barrier()
        pltpu.sync_copy(acc.at[pl.ds(r0, _RPT)], initb)
        pltpu.sync_copy(initb, out_hbm.at[pl.ds(cid * _NP + r0, _RPT)])

    return k


def _agg_build(F, shift):
    @functools.partial(
        pl.kernel,
        out_type=jax.ShapeDtypeStruct((2, _NP, 128), jnp.float32),
        mesh=_mesh(),
        compiler_params=_SC_PARAMS,
        scratch_types=[
            pltpu.VMEM_SHARED((_NP, F), jnp.float32),
            pltpu.VMEM((8, _CHUNK), jnp.int32),
            pltpu.VMEM((8, _CHUNK), jnp.int32),
            pltpu.VMEM((4, _CHUNK, F), jnp.float32),
            pltpu.VMEM((_IC, F), jnp.float32),
            pltpu.SemaphoreType.DMA((8,)),
            pltpu.SemaphoreType.DMA((8,)),
            pltpu.SemaphoreType.DMA((4,)),
            pltpu.SemaphoreType.DMA((4,)),
        ],
    )
    def k(ep_hbm, yv_hbm, out_hbm, acc,
          sidx, didx, rows, zbuf, ssem, dsem, gsem, wsem):
        cid = lax.axis_index("c")
        sid = lax.axis_index("s")
        wid = cid * 16 + sid
        base = wid * _CPT
        row0 = sid * _RPT

        zero16 = jnp.zeros((16,), jnp.float32)

        @pl.loop(0, _IC)
        def _(i):
            for c in range(F // 16):
                zbuf[i, pl.ds(c * 16, 16)] = zero16

        @pl.loop(0, _RPT, step=_IC)
        def _(i):
            pltpu.sync_copy(zbuf, acc.at[pl.ds(row0 + i, _IC)])

        plsc.subcore_barrier()

        def issue_idx(g, b):
            off = (base + g) * _CHUNK
            pltpu.async_copy(ep_hbm.at[0, pl.ds(off, _CHUNK)], sidx.at[b],
                             ssem.at[b])
            pltpu.async_copy(ep_hbm.at[1, pl.ds(off, _CHUNK)], didx.at[b],
                             dsem.at[b])

        def wait_idx(g, b):
            off = (base + g) * _CHUNK
            pltpu.make_async_copy(ep_hbm.at[0, pl.ds(off, _CHUNK)],
                                  sidx.at[b], ssem.at[b]).wait()
            pltpu.make_async_copy(ep_hbm.at[1, pl.ds(off, _CHUNK)],
                                  didx.at[b], dsem.at[b]).wait()
            for j in range(_CHUNK // 16):
                sl = pl.ds(j * 16, 16)
                sidx[b, sl] = sidx[b, sl] << shift

        def issue_gather(bi, br):
            pltpu.async_copy(yv_hbm.at[sidx.at[bi]], rows.at[br], gsem.at[br])

        def wait_gather(bi, br):
            pltpu.make_async_copy(yv_hbm.at[sidx.at[bi]], rows.at[br],
                                  gsem.at[br]).wait()

        def issue_scatter(bi, br):
            pltpu.async_copy(rows.at[br], acc.at[didx.at[bi]], wsem.at[br],
                             add=True)

        def wait_scatter(bi, br):
            pltpu.make_async_copy(rows.at[br], acc.at[didx.at[bi]],
                                  wsem.at[br]).wait()

        for g0 in range(6):
            issue_idx(g0, g0)
        wait_idx(0, 0)
        issue_gather(0, 0)
        wait_idx(1, 1)
        issue_gather(1, 1)

        @pl.loop(0, _CPT, step=8)
        def _(g):
            for db in range(8):
                gg = g + db
                b8 = db % 8
                b4 = db % 4

                @pl.when(gg >= 2)
                def _():
                    wait_scatter((db + 6) % 8, (db + 2) % 4)

                @pl.when(gg + 6 < _CPT)
                def _():
                    issue_idx(gg + 6, (db + 6) % 8)

                @pl.when(gg + 2 < _CPT)
                def _():
                    wait_idx(gg + 2, (db + 2) % 8)
                    issue_gather((db + 2) % 8, (db + 2) % 4)

                wait_gather(b8, b4)
                issue_scatter(b8, b4)

        wait_scatter(6, 2)
        wait_scatter(7, 3)

        plsc.subcore_barrier()

        @pl.loop(0, _RPT, step=_IC)
        def _(i):
            pltpu.sync_copy(acc.at[pl.ds(row0 + i, _IC)],
                            rows.at[0, pl.ds(0, _IC)])
            pltpu.sync_copy(rows.at[0, pl.ds(0, _IC)],
                            out_hbm.at[cid, pl.ds(row0 + i, _IC), pl.ds(0, F)])

    return k


def _pool_pair_build():
    ppt = _P // _NT
    cols = _GB // 16

    @functools.partial(
        pl.kernel,
        out_type=jax.ShapeDtypeStruct((_P,), jnp.float32),
        mesh=_mesh(),
        compiler_params=_SC_PARAMS_NL,
        scratch_types=[
            pltpu.VMEM_SHARED((16, 2 * _GB), jnp.float32),
            pltpu.VMEM_SHARED((_GB,), jnp.float32),
            pltpu.VMEM((2 * _GB,), jnp.float32),
            pltpu.VMEM((_IC,), jnp.float32),
            pltpu.VMEM((_IC,), jnp.int32),
            pltpu.VMEM((16, cols), jnp.float32),
            pltpu.VMEM((_G,), jnp.float32),
            pltpu.VMEM((ppt,), jnp.int32),
            pltpu.VMEM((ppt,), jnp.int32),
            pltpu.VMEM((ppt,), jnp.float32),
            pltpu.SemaphoreType.DMA,
        ],
    )
    def k(s_hbm, batch_hbm, ia_hbm, ib_hbm, out_hbm,
          stage, ushared, hloc, sv, bv, comb, ubuf, av, bv2, ov, sem):
        cid = lax.axis_index("c")
        sid = lax.axis_index("s")
        wid = cid * 16 + sid
        r0 = sid * _RPT

        zero16 = jnp.zeros((16,), jnp.float32)
        one16 = jnp.full((16,), 1.0, jnp.float32)

        @pl.loop(0, 2 * _GB, step=16)
        def _(i):
            hloc[pl.ds(i, 16)] = zero16

        @pl.loop(0, _RPT, step=_IC)
        def _(i):
            pltpu.sync_copy(s_hbm.at[pl.ds(r0 + i, _IC)], sv)
            pltpu.sync_copy(batch_hbm.at[pl.ds(r0 + i, _IC)], bv)

            @pl.loop(0, _IC, step=16)
            def _(j):
                b16 = bv[pl.ds(j, 16)]
                plsc.addupdate_scatter(hloc, [b16], sv[pl.ds(j, 16)])
                plsc.addupdate_scatter(hloc, [b16 + _GB], one16)

        pltpu.sync_copy(hloc, stage.at[sid])
        plsc.subcore_barrier()

        c0 = sid * cols
        pltpu.sync_copy(stage.at[pl.ds(0, 16), pl.ds(c0, cols)], comb)

        @pl.loop(0, cols, step=16)
        def _(j):
            t = comb[0, pl.ds(j, 16)]
            for r in range(1, 16):
                t = t + comb[r, pl.ds(j, 16)]
            hloc[pl.ds(j, 16)] = t

        pltpu.sync_copy(stage.at[pl.ds(0, 16), pl.ds(_GB + c0, cols)], comb)

        @pl.loop(0, cols, step=16)
        def _(j):
            t = comb[0, pl.ds(j, 16)]
            for r in range(1, 16):
                t = t + comb[r, pl.ds(j, 16)]
            hloc[pl.ds(j, 16)] = hloc[pl.ds(j, 16)] / jnp.maximum(t, one16)

        pltpu.sync_copy(hloc.at[pl.ds(0, cols)], ushared.at[pl.ds(c0, cols)])
        plsc.subcore_barrier()

        pltpu.sync_copy(ushared.at[pl.ds(0, _G)], ubuf)

        p0 = wid * ppt
        pltpu.sync_copy(ia_hbm.at[pl.ds(p0, ppt)], av)
        pltpu.sync_copy(ib_hbm.at[pl.ds(p0, ppt)], bv2)

        @pl.loop(0, ppt, step=16)
        def _(i):
            sl = pl.ds(i, 16)
            ua = plsc.load_gather(ubuf, [av[sl]])
            ub = plsc.load_gather(ubuf, [bv2[sl]])
            ov[sl] = ub - ua

        pltpu.sync_copy(ov, out_hbm.at[pl.ds(p0, ppt)])

    return k


def _t1_body(da_ref, db_ref, x_ref, dinv_ref, y1_ref):
    deg = da_ref[...] + db_ref[...] - 1.0
    dinv = lax.rsqrt(deg)
    dinv_ref[...] = dinv
    y1_ref[...] = x_ref[...] * dinv.reshape(_BLK, 1)


def _t1(parts, x_pk):
    return pl.pallas_call(
        _t1_body,
        grid=(_NP // _BLK,),
        in_specs=[
            pl.BlockSpec((_BLK,), lambda i: (i,)),
            pl.BlockSpec((_BLK,), lambda i: (i + _NP // _BLK,)),
            pl.BlockSpec((_BLK, 128), lambda i: (i, 0)),
        ],
        out_specs=[
            pl.BlockSpec((_BLK,), lambda i: (i,)),
            pl.BlockSpec((_BLK, 128), lambda i: (i, 0)),
        ],
        out_shape=[
            jax.ShapeDtypeStruct((_NP,), jnp.float32),
            jax.ShapeDtypeStruct((_NP, 128), jnp.float32),
        ],
    )(parts, parts, x_pk)


def _t2_body(p_ref, y1_ref, dinv_ref, w_ref, b_ref, y2_ref):
    dinv = dinv_ref[...].reshape(_BLK, 1)
    agg = p_ref[0] + p_ref[1] + y1_ref[...]
    z = agg[:, :16] * dinv
    h = jnp.tanh(
        jax.lax.dot_general(z, w_ref[...], (((1,), (0,)), ((), ())),
                            precision=lax.Precision.HIGHEST,
                            preferred_element_type=jnp.float32)
        + b_ref[...])
    y2_ref[...] = jnp.concatenate(
        [h * dinv, jnp.zeros((_BLK, 96), jnp.float32)], axis=1)


def _t2(parts, y1, dinv, W1p, b1):
    return pl.pallas_call(
        _t2_body,
        grid=(_NP // _BLK,),
        in_specs=[
            pl.BlockSpec((2, _BLK, 128), lambda i: (0, i, 0)),
            pl.BlockSpec((_BLK, 128), lambda i: (i, 0)),
            pl.BlockSpec((_BLK,), lambda i: (i,)),
            pl.BlockSpec((16, 32), lambda i: (0, 0)),
            pl.BlockSpec((1, 32), lambda i: (0, 0)),
        ],
        out_specs=pl.BlockSpec((_BLK, 128), lambda i: (i, 0)),
        out_shape=jax.ShapeDtypeStruct((_NP, 128), jnp.float32),
    )(parts, y1, dinv, W1p, b1)


def _t3_body(p_ref, y2_ref, dinv_ref, w_ref, b_ref, f1w_ref, fw_ref, s_ref):
    dinv = dinv_ref[...].reshape(_BLK, 1)
    agg = p_ref[0] + p_ref[1] + y2_ref[...]
    z = agg[:, :32] * dinv
    h = jnp.tanh(
        jax.lax.dot_general(z, w_ref[...], (((1,), (0,)), ((), ())),
                            precision=lax.Precision.HIGHEST,
                            preferred_element_type=jnp.float32)
        + b_ref[...])
    vrow = jax.lax.dot_general(fw_ref[...], f1w_ref[...],
                               (((0,), (1,)), ((), ())),
                               precision=lax.Precision.HIGHEST,
                               preferred_element_type=jnp.float32)
    s_ref[...] = jnp.sum(h * vrow, axis=1)


def _t3(parts, y2, dinv, W2, b2, fc1_W, fc_W):
    return pl.pallas_call(
        _t3_body,
        grid=(_NP // _BLK,),
        in_specs=[
            pl.BlockSpec((2, _BLK, 128), lambda i: (0, i, 0)),
            pl.BlockSpec((_BLK, 128), lambda i: (i, 0)),
            pl.BlockSpec((_BLK,), lambda i: (i,)),
            pl.BlockSpec((32, 32), lambda i: (0, 0)),
            pl.BlockSpec((1, 32), lambda i: (0, 0)),
            pl.BlockSpec((32, 32), lambda i: (0, 0)),
            pl.BlockSpec((32, 1), lambda i: (0, 0)),
        ],
        out_specs=pl.BlockSpec((_BLK,), lambda i: (i,)),
        out_shape=jax.ShapeDtypeStruct((_NP,), jnp.float32),
    )(parts, y2, dinv, W2, b2, fc1_W, fc_W)


def kernel(x, edge_index, batch, idx_a, idx_b, W1, b1, W2, b2,
           fc1_W, fc1_b, fc_W, fc_b):
    ep = lax.pad(edge_index, jnp.int32(_N), ((0, 0, 0), (0, _EP - _E, 0)))
    x_pk = jnp.pad(x, ((0, _NP - _N), (0, 128 - x.shape[1])))
    batch_pad = jnp.pad(batch, (0, _NP - _N), constant_values=_G)
    W1p = jnp.pad(W1, ((0, 16 - W1.shape[0]), (0, 0)))

    deg_parts = _deg_build()(ep)
    dinv, y1 = _t1(deg_parts, x_pk)
    p1 = _agg_build(16, 3)(ep, y1.reshape(_NP * 8, 16))
    y2 = _t2(p1, y1, dinv, W1p, b1.reshape(1, 32))
    p2 = _agg_build(32, 2)(ep, y2.reshape(_NP * 4, 32))
    s = _t3(p2, y2, dinv, W2, b2.reshape(1, 32), fc1_W, fc_W)
    return _pool_pair_build()(s, batch_pad, idx_a, idx_b)
